# baseline (speedup 1.0000x reference)
import jax
import jax.numpy as jnp
from jax import lax
from jax.experimental import pallas as pl
from jax.experimental.pallas import tpu as pltpu

_B, _IN, _OUT = 2048, 128, 128
_F32 = jnp.float32
_HI = lax.Precision.HIGHEST
_CPB = 16


def _roll0(a, s):
    return jnp.roll(a, s, axis=0)


def _bitonic_desc(arrs, cmp_first):
    n = 128
    row = lax.broadcasted_iota(jnp.int32, (n, n), 0)
    k = 2
    while k <= n:
        d = k // 2
        while d >= 1:
            hi = (row & d) != 0
            partner = tuple(
                jnp.where(hi, _roll0(a, d), _roll0(a, -d)) for a in arrs)
            desc = (row & k) == 0
            keep_first = jnp.logical_xor(desc, hi)
            self_first = cmp_first(arrs, partner)
            take_self = self_first == keep_first
            arrs = tuple(
                jnp.where(take_self, a, p) for a, p in zip(arrs, partner))
            d //= 2
        k *= 2
    return arrs


def _loss_body(x_ref, t_ref, w_ref, out_ref, sc_ref):
    x = x_ref[...]
    t = t_ref[...]
    w = w_ref[...]

    lane_iota = lax.broadcasted_iota(jnp.int32, (_CPB, _IN), 1)
    sub_iota = lax.broadcasted_iota(jnp.int32, (_CPB, _IN), 0)

    _CH = 128

    def arow(g, carry):
        i0 = g * _CPB
        e = (lane_iota == (i0 + sub_iota)).astype(_F32)
        xcols = lax.dot_general(x, e, ((((1,), (1,))), ((), ())),
                                preferred_element_type=_F32,
                                precision=_HI)
        dcols = jnp.sum(xcols, axis=0, keepdims=True)
        accs = [jnp.zeros((8, _OUT), _F32) for _ in range(_CPB)]
        for c in range(_B // _CH):
            tc = t[c * _CH:(c + 1) * _CH, :]
            xch = xcols[c * _CH:(c + 1) * _CH, :]
            for kk in range(_CPB):
                xcc = xch[:, kk:kk + 1]
                m = jnp.minimum(tc, xcc)
                part = jnp.sum(m.reshape(_CH // 8, 8, _OUT), axis=0)
                accs[kk] = accs[kk] + part
        for kk in range(_CPB):
            row = jnp.sum(accs[kk], axis=0, keepdims=True)
            srow = row * (1.0 / dcols[0:1, kk:kk + 1])
            srow = jnp.where(jnp.isnan(srow), 1.0, srow)
            sc_ref[pl.ds(i0 + kk, 1), :] = srow
        return carry

    lax.fori_loop(0, _IN // _CPB, arow, 0)
    score = sc_ref[...]

    idx0 = lax.broadcasted_iota(jnp.int32, (_IN, _OUT), 0)

    def cmp_score(s, o):
        sk, si, _ = s
        ok, oi, _ = o
        return (sk > ok) | ((sk == ok) & (si < oi))

    _, _, tw = _bitonic_desc((score, idx0, w), cmp_score)

    def cmp_w(s, o):
        return s[0] > o[0]

    (sw,) = _bitonic_desc((w,), cmp_w)

    diff = sw - tw
    loss = jnp.sum(diff * diff) / (_IN * _OUT)
    out_ref[...] = jnp.broadcast_to(loss, (1, 1))


def kernel(x, y, t, w, base_w):
    del y, base_w
    out = pl.pallas_call(
        _loss_body,
        out_shape=jax.ShapeDtypeStruct((1, 1), _F32),
        scratch_shapes=[pltpu.VMEM((_IN, _OUT), _F32)],
    )(x, t, w)
    return out[0, 0]


# speedup vs baseline: 1.0799x; 1.0799x over previous
import jax
import jax.numpy as jnp
from jax import lax
from jax.experimental import pallas as pl
from jax.experimental.pallas import tpu as pltpu

_B, _IN, _OUT = 2048, 128, 128
_F32 = jnp.float32
_HI = lax.Precision.HIGHEST
_CPB = 16


def _roll0(a, s):
    return jnp.roll(a, s, axis=0)


def _bitonic_desc(arrs, cmp_first):
    n = 128
    row = lax.broadcasted_iota(jnp.int32, (n, n), 0)
    k = 2
    while k <= n:
        d = k // 2
        while d >= 1:
            hi = (row & d) != 0
            partner = tuple(
                jnp.where(hi, _roll0(a, d), _roll0(a, -d)) for a in arrs)
            desc = (row & k) == 0
            keep_first = jnp.logical_xor(desc, hi)
            self_first = cmp_first(arrs, partner)
            take_self = self_first == keep_first
            arrs = tuple(
                jnp.where(take_self, a, p) for a, p in zip(arrs, partner))
            d //= 2
        k *= 2
    return arrs


def _loss_body(x_ref, t_ref, w_ref, out_ref, sc_ref):
    x = x_ref[...]
    t = t_ref[...]
    w = w_ref[...]

    lane_iota = lax.broadcasted_iota(jnp.int32, (_CPB, _IN), 1)
    sub_iota = lax.broadcasted_iota(jnp.int32, (_CPB, _IN), 0)

    ones_row = jnp.ones((1, _B), _F32)
    d_all = lax.dot_general(ones_row, x, ((((1,), (0,))), ((), ())),
                            preferred_element_type=_F32,
                            precision=_HI)

    def arow(g, carry):
        i0 = g * _CPB
        e = (lane_iota == (i0 + sub_iota)).astype(_F32)
        xcols = lax.dot_general(x, e, ((((1,), (1,))), ((), ())),
                                preferred_element_type=_F32,
                                precision=_HI)
        dcols = lax.dot_general(d_all, e, ((((1,), (1,))), ((), ())),
                                preferred_element_type=_F32,
                                precision=_HI)
        for kk in range(_CPB):
            xc = xcols[:, kk:kk + 1]
            m = jnp.minimum(t, xc)
            row = jnp.sum(m, axis=0, keepdims=True)
            srow = row * (1.0 / dcols[0:1, kk:kk + 1])
            srow = jnp.where(jnp.isnan(srow), 1.0, srow)
            sc_ref[pl.ds(i0 + kk, 1), :] = srow
        return carry

    lax.fori_loop(0, _IN // _CPB, arow, 0)
    score = sc_ref[...]

    idx0 = lax.broadcasted_iota(jnp.int32, (_IN, _OUT), 0)

    def cmp_score(s, o):
        sk, si, _ = s
        ok, oi, _ = o
        return (sk > ok) | ((sk == ok) & (si < oi))

    _, _, tw = _bitonic_desc((score, idx0, w), cmp_score)

    def cmp_w(s, o):
        return s[0] > o[0]

    (sw,) = _bitonic_desc((w,), cmp_w)

    diff = sw - tw
    loss = jnp.sum(diff * diff) / (_IN * _OUT)
    out_ref[...] = jnp.broadcast_to(loss, (1, 1))


def kernel(x, y, t, w, base_w):
    del y, base_w
    out = pl.pallas_call(
        _loss_body,
        out_shape=jax.ShapeDtypeStruct((1, 1), _F32),
        scratch_shapes=[pltpu.VMEM((_IN, _OUT), _F32)],
    )(x, t, w)
    return out[0, 0]


# speedup vs baseline: 1.0898x; 1.0092x over previous
import jax
import jax.numpy as jnp
from jax import lax
from jax.experimental import pallas as pl
from jax.experimental.pallas import tpu as pltpu

_B, _IN, _OUT = 2048, 128, 128
_F32 = jnp.float32
_HI = lax.Precision.HIGHEST
_CPB = 16


def _roll0(a, s):
    return jnp.roll(a, s, axis=0)


def _bitonic_desc(arrs, cmp_first):
    n = 128
    row = lax.broadcasted_iota(jnp.int32, (n, n), 0)
    k = 2
    while k <= n:
        d = k // 2
        while d >= 1:
            hi = (row & d) != 0
            partner = tuple(
                jnp.where(hi, _roll0(a, d), _roll0(a, -d)) for a in arrs)
            desc = (row & k) == 0
            keep_first = jnp.logical_xor(desc, hi)
            self_first = cmp_first(arrs, partner)
            take_self = self_first == keep_first
            arrs = tuple(
                jnp.where(take_self, a, p) for a, p in zip(arrs, partner))
            d //= 2
        k *= 2
    return arrs


def _loss_body(x_ref, t_ref, w_ref, out_ref, sc_ref):
    x = x_ref[...]
    t = t_ref[...]
    w = w_ref[...]

    lane_iota = lax.broadcasted_iota(jnp.int32, (_CPB, _IN), 1)
    sub_iota = lax.broadcasted_iota(jnp.int32, (_CPB, _IN), 0)

    def arow(g, carry):
        i0 = g * _CPB
        e = (lane_iota == (i0 + sub_iota)).astype(_F32)
        xcols = lax.dot_general(x, e, ((((1,), (1,))), ((), ())),
                                preferred_element_type=_F32,
                                precision=_HI)
        dcols = jnp.sum(xcols, axis=0, keepdims=True)
        for kk in range(_CPB):
            xc = xcols[:, kk:kk + 1]
            m = jnp.minimum(t, xc)
            row = jnp.sum(m, axis=0, keepdims=True)
            srow = row * (1.0 / dcols[0:1, kk:kk + 1])
            srow = jnp.where(jnp.isnan(srow), 1.0, srow)
            sc_ref[pl.ds(i0 + kk, 1), :] = srow
        return carry

    lax.fori_loop(0, _IN // _CPB, arow, 0)
    score = sc_ref[...]

    idx0 = lax.broadcasted_iota(jnp.int32, (_IN, _OUT), 0)

    def cmp_score(s, o):
        sk, si, _ = s
        ok, oi, _ = o
        return (sk > ok) | ((sk == ok) & (si < oi))

    _, _, tw = _bitonic_desc((score, idx0, w), cmp_score)

    def cmp_w(s, o):
        return s[0] > o[0]

    (sw,) = _bitonic_desc((w,), cmp_w)

    diff = sw - tw
    loss = jnp.sum(diff * diff) / (_IN * _OUT)
    out_ref[...] = jnp.broadcast_to(loss, (1, 1))


def kernel(x, y, t, w, base_w):
    del y, base_w
    out = pl.pallas_call(
        _loss_body,
        out_shape=jax.ShapeDtypeStruct((1, 1), _F32),
        scratch_shapes=[pltpu.VMEM((_IN, _OUT), _F32)],
    )(x, t, w)
    return out[0, 0]


# speedup vs baseline: 1.1863x; 1.0885x over previous
import jax
import jax.numpy as jnp
from jax import lax
from jax.experimental import pallas as pl
from jax.experimental.pallas import tpu as pltpu

_B, _IN, _OUT = 2048, 128, 128
_F32 = jnp.float32
_HI = lax.Precision.HIGHEST
_CPB = 32


def _roll0(a, s):
    return jnp.roll(a, s, axis=0)


def _bitonic_desc(arrs, cmp_first):
    n = 128
    row = lax.broadcasted_iota(jnp.int32, (n, n), 0)
    k = 2
    while k <= n:
        d = k // 2
        while d >= 1:
            hi = (row & d) != 0
            partner = tuple(
                jnp.where(hi, _roll0(a, d), _roll0(a, -d)) for a in arrs)
            desc = (row & k) == 0
            keep_first = jnp.logical_xor(desc, hi)
            self_first = cmp_first(arrs, partner)
            take_self = self_first == keep_first
            arrs = tuple(
                jnp.where(take_self, a, p) for a, p in zip(arrs, partner))
            d //= 2
        k *= 2
    return arrs


def _loss_body(x_ref, t_ref, w_ref, out_ref, sc_ref):
    x = x_ref[...]
    t = t_ref[...]
    w = w_ref[...]

    lane_iota = lax.broadcasted_iota(jnp.int32, (_CPB, _IN), 1)
    sub_iota = lax.broadcasted_iota(jnp.int32, (_CPB, _IN), 0)

    def arow(g, carry):
        i0 = g * _CPB
        e = (lane_iota == (i0 + sub_iota)).astype(_F32)
        xcols = lax.dot_general(x, e, ((((1,), (1,))), ((), ())),
                                preferred_element_type=_F32,
                                precision=_HI)
        dcols = jnp.sum(xcols, axis=0, keepdims=True)
        for kk in range(_CPB):
            xc = xcols[:, kk:kk + 1]
            m = jnp.minimum(t, xc)
            row = jnp.sum(m, axis=0, keepdims=True)
            srow = row * (1.0 / dcols[0:1, kk:kk + 1])
            srow = jnp.where(jnp.isnan(srow), 1.0, srow)
            sc_ref[pl.ds(i0 + kk, 1), :] = srow
        return carry

    lax.fori_loop(0, _IN // _CPB, arow, 0)
    score = sc_ref[...]

    idx0 = lax.broadcasted_iota(jnp.int32, (_IN, _OUT), 0)

    def cmp_score(s, o):
        sk, si, _ = s
        ok, oi, _ = o
        return (sk > ok) | ((sk == ok) & (si < oi))

    _, _, tw = _bitonic_desc((score, idx0, w), cmp_score)

    def cmp_w(s, o):
        return s[0] > o[0]

    (sw,) = _bitonic_desc((w,), cmp_w)

    diff = sw - tw
    loss = jnp.sum(diff * diff) / (_IN * _OUT)
    out_ref[...] = jnp.broadcast_to(loss, (1, 1))


def kernel(x, y, t, w, base_w):
    del y, base_w
    out = pl.pallas_call(
        _loss_body,
        out_shape=jax.ShapeDtypeStruct((1, 1), _F32),
        scratch_shapes=[pltpu.VMEM((_IN, _OUT), _F32)],
    )(x, t, w)
    return out[0, 0]


# speedup vs baseline: 1.2425x; 1.0474x over previous
import jax
import jax.numpy as jnp
from jax import lax
from jax.experimental import pallas as pl
from jax.experimental.pallas import tpu as pltpu

_B, _IN, _OUT = 2048, 128, 128
_F32 = jnp.float32
_HI = lax.Precision.HIGHEST
_CPB = 64


def _roll0(a, s):
    return jnp.roll(a, s, axis=0)


def _bitonic_desc(arrs, cmp_first):
    n = 128
    row = lax.broadcasted_iota(jnp.int32, (n, n), 0)
    k = 2
    while k <= n:
        d = k // 2
        while d >= 1:
            hi = (row & d) != 0
            partner = tuple(
                jnp.where(hi, _roll0(a, d), _roll0(a, -d)) for a in arrs)
            desc = (row & k) == 0
            keep_first = jnp.logical_xor(desc, hi)
            self_first = cmp_first(arrs, partner)
            take_self = self_first == keep_first
            arrs = tuple(
                jnp.where(take_self, a, p) for a, p in zip(arrs, partner))
            d //= 2
        k *= 2
    return arrs


def _loss_body(x_ref, t_ref, w_ref, out_ref, sc_ref):
    x = x_ref[...]
    t = t_ref[...]
    w = w_ref[...]

    lane_iota = lax.broadcasted_iota(jnp.int32, (_CPB, _IN), 1)
    sub_iota = lax.broadcasted_iota(jnp.int32, (_CPB, _IN), 0)

    def arow(g, carry):
        i0 = g * _CPB
        e = (lane_iota == (i0 + sub_iota)).astype(_F32)
        xcols = lax.dot_general(x, e, ((((1,), (1,))), ((), ())),
                                preferred_element_type=_F32,
                                precision=_HI)
        dcols = jnp.sum(xcols, axis=0, keepdims=True)
        for kk in range(_CPB):
            xc = xcols[:, kk:kk + 1]
            m = jnp.minimum(t, xc)
            row = jnp.sum(m, axis=0, keepdims=True)
            srow = row * (1.0 / dcols[0:1, kk:kk + 1])
            srow = jnp.where(jnp.isnan(srow), 1.0, srow)
            sc_ref[pl.ds(i0 + kk, 1), :] = srow
        return carry

    lax.fori_loop(0, _IN // _CPB, arow, 0)
    score = sc_ref[...]

    idx0 = lax.broadcasted_iota(jnp.int32, (_IN, _OUT), 0)

    def cmp_score(s, o):
        sk, si, _ = s
        ok, oi, _ = o
        return (sk > ok) | ((sk == ok) & (si < oi))

    _, _, tw = _bitonic_desc((score, idx0, w), cmp_score)

    def cmp_w(s, o):
        return s[0] > o[0]

    (sw,) = _bitonic_desc((w,), cmp_w)

    diff = sw - tw
    loss = jnp.sum(diff * diff) / (_IN * _OUT)
    out_ref[...] = jnp.broadcast_to(loss, (1, 1))


def kernel(x, y, t, w, base_w):
    del y, base_w
    out = pl.pallas_call(
        _loss_body,
        out_shape=jax.ShapeDtypeStruct((1, 1), _F32),
        scratch_shapes=[pltpu.VMEM((_IN, _OUT), _F32)],
    )(x, t, w)
    return out[0, 0]


# speedup vs baseline: 1.2563x; 1.0111x over previous
import jax
import jax.numpy as jnp
from jax import lax
from jax.experimental import pallas as pl
from jax.experimental.pallas import tpu as pltpu

_B, _IN, _OUT = 2048, 128, 128
_F32 = jnp.float32
_HI = lax.Precision.HIGHEST
_CPB = 128


def _roll0(a, s):
    return jnp.roll(a, s, axis=0)


def _bitonic_desc(arrs, cmp_first):
    n = 128
    row = lax.broadcasted_iota(jnp.int32, (n, n), 0)
    k = 2
    while k <= n:
        d = k // 2
        while d >= 1:
            hi = (row & d) != 0
            partner = tuple(
                jnp.where(hi, _roll0(a, d), _roll0(a, -d)) for a in arrs)
            desc = (row & k) == 0
            keep_first = jnp.logical_xor(desc, hi)
            self_first = cmp_first(arrs, partner)
            take_self = self_first == keep_first
            arrs = tuple(
                jnp.where(take_self, a, p) for a, p in zip(arrs, partner))
            d //= 2
        k *= 2
    return arrs


def _loss_body(x_ref, t_ref, w_ref, out_ref, sc_ref):
    x = x_ref[...]
    t = t_ref[...]
    w = w_ref[...]

    lane_iota = lax.broadcasted_iota(jnp.int32, (_CPB, _IN), 1)
    sub_iota = lax.broadcasted_iota(jnp.int32, (_CPB, _IN), 0)

    def arow(g, carry):
        i0 = g * _CPB
        e = (lane_iota == (i0 + sub_iota)).astype(_F32)
        xcols = lax.dot_general(x, e, ((((1,), (1,))), ((), ())),
                                preferred_element_type=_F32,
                                precision=_HI)
        dcols = jnp.sum(xcols, axis=0, keepdims=True)
        for kk in range(_CPB):
            xc = xcols[:, kk:kk + 1]
            m = jnp.minimum(t, xc)
            row = jnp.sum(m, axis=0, keepdims=True)
            srow = row * (1.0 / dcols[0:1, kk:kk + 1])
            srow = jnp.where(jnp.isnan(srow), 1.0, srow)
            sc_ref[pl.ds(i0 + kk, 1), :] = srow
        return carry

    lax.fori_loop(0, _IN // _CPB, arow, 0)
    score = sc_ref[...]

    idx0 = lax.broadcasted_iota(jnp.int32, (_IN, _OUT), 0)

    def cmp_score(s, o):
        sk, si, _ = s
        ok, oi, _ = o
        return (sk > ok) | ((sk == ok) & (si < oi))

    _, _, tw = _bitonic_desc((score, idx0, w), cmp_score)

    def cmp_w(s, o):
        return s[0] > o[0]

    (sw,) = _bitonic_desc((w,), cmp_w)

    diff = sw - tw
    loss = jnp.sum(diff * diff) / (_IN * _OUT)
    out_ref[...] = jnp.broadcast_to(loss, (1, 1))


def kernel(x, y, t, w, base_w):
    del y, base_w
    out = pl.pallas_call(
        _loss_body,
        out_shape=jax.ShapeDtypeStruct((1, 1), _F32),
        scratch_shapes=[pltpu.VMEM((_IN, _OUT), _F32)],
    )(x, t, w)
    return out[0, 0]
